# baseline (speedup 1.0000x reference)
import functools

import jax
import jax.numpy as jnp
from jax import lax
from jax.experimental import pallas as pl
from jax.experimental.pallas import tpu as pltpu
from jax.experimental.pallas import tpu_sc as plsc

GAMMA = 12.0

NUM_ENTS = 1000000
DIM = 64
BATCH = 4096
NEG = 256

NUM_WORKERS = 32
BPW = BATCH // NUM_WORKERS
CHUNK = 2
NCHUNKS = BPW // CHUNK
ROWS_PER_CHUNK = CHUNK * NEG


def _sc_body(x_hbm, tgt_hbm, pos_hbm, neg_hbm, ent_hbm,
             pos_out, neg_out,
             tgt_v, posidx_v, negidx_v, pred_v, posrow_v, negrow_v,
             posout_v, negout_v):
  nc = 2
  wid = lax.axis_index("s") * nc + lax.axis_index("c")
  base = wid * BPW
  iota = lax.iota(jnp.int32, 16)

  pltpu.sync_copy(tgt_hbm.at[pl.ds(base, BPW)], tgt_v)
  pltpu.sync_copy(pos_hbm.at[pl.ds(base, BPW)], posidx_v)
  pltpu.sync_copy(x_hbm.at[tgt_v], pred_v)
  pltpu.sync_copy(ent_hbm.at[posidx_v], posrow_v)

  def pos_group(g, carry):
    rows = g * 16 + iota
    acc = jnp.zeros((16,), jnp.float32)
    for d in range(DIM):
      col = jnp.full((16,), d, jnp.int32)
      ev = plsc.load_gather(posrow_v, [rows, col])
      pv = plsc.load_gather(pred_v, [rows, col])
      acc = acc + jnp.abs(ev - pv)
    posout_v[pl.ds(g * 16, 16)] = GAMMA - acc
    return carry

  lax.fori_loop(0, BPW // 16, pos_group, 0)
  pltpu.sync_copy(posout_v, pos_out.at[pl.ds(base, BPW)])

  def chunk_body(c, carry):
    flat_base = (base + c * CHUNK) * NEG
    pltpu.sync_copy(neg_hbm.at[pl.ds(flat_base, ROWS_PER_CHUNK)], negidx_v)
    for j in range(ROWS_PER_CHUNK // 128):
      pltpu.sync_copy(ent_hbm.at[negidx_v.at[pl.ds(j * 128, 128)]],
                      negrow_v.at[pl.ds(j * 128, 128), :])

    for bb in range(CHUNK):
      prow = c * CHUNK + bb
      pvecs = [pred_v[prow, pl.ds(k * 16, 16)] for k in range(DIM // 16)]

      def neg_group(g, carry2, bb=bb, pvecs=pvecs):
        rows = bb * NEG + g * 16 + iota
        acc = jnp.zeros((16,), jnp.float32)
        for d in range(DIM):
          col = jnp.full((16,), d, jnp.int32)
          ev = plsc.load_gather(negrow_v, [rows, col])
          p = pvecs[d // 16][d % 16]
          acc = acc + jnp.abs(ev - p)
        negout_v[pl.ds(bb * NEG + g * 16, 16)] = GAMMA - acc
        return carry2

      lax.fori_loop(0, NEG // 16, neg_group, 0)

    pltpu.sync_copy(negout_v, neg_out.at[pl.ds(flat_base, ROWS_PER_CHUNK)])
    return carry

  lax.fori_loop(0, NCHUNKS, chunk_body, 0)


@jax.jit
def _sc_kernel(x, tgt, pos, neg_flat, ent):
  mesh = plsc.VectorSubcoreMesh(core_axis_name="c", subcore_axis_name="s")
  f = functools.partial(
      pl.kernel,
      mesh=mesh,
      compiler_params=pltpu.CompilerParams(
          needs_layout_passes=False, use_tc_tiling_on_sc=False),
      out_type=(
          jax.ShapeDtypeStruct((BATCH,), jnp.float32),
          jax.ShapeDtypeStruct((BATCH * NEG,), jnp.float32),
      ),
      scratch_types=[
          pltpu.VMEM((BPW,), jnp.int32),
          pltpu.VMEM((BPW,), jnp.int32),
          pltpu.VMEM((ROWS_PER_CHUNK,), jnp.int32),
          pltpu.VMEM((BPW, DIM), jnp.float32),
          pltpu.VMEM((BPW, DIM), jnp.float32),
          pltpu.VMEM((ROWS_PER_CHUNK, DIM), jnp.float32),
          pltpu.VMEM((BPW,), jnp.float32),
          pltpu.VMEM((ROWS_PER_CHUNK,), jnp.float32),
      ],
  )(_sc_body)
  return f(x, tgt, pos, neg_flat, ent)


def kernel(x, target_node_idxes, positive_samples, negative_samples,
           ent_embedding):
  tgt = target_node_idxes.astype(jnp.int32)
  pos = positive_samples.astype(jnp.int32)
  neg_flat = negative_samples.astype(jnp.int32).reshape(-1)
  pos_l, neg_l = _sc_kernel(x, tgt, pos, neg_flat, ent_embedding)
  return pos_l[:, None], neg_l.reshape(BATCH, NEG)


# speedup vs baseline: 2.3470x; 2.3470x over previous
import functools

import jax
import jax.numpy as jnp
from jax import lax
from jax.experimental import pallas as pl
from jax.experimental.pallas import tpu as pltpu
from jax.experimental.pallas import tpu_sc as plsc

GAMMA = 12.0

NUM_ENTS = 1000000
DIM = 64
BATCH = 4096
NEG = 256

NUM_WORKERS = 32
BPW = BATCH // NUM_WORKERS
CHUNK = 2
NCHUNKS = BPW // CHUNK
RPC = CHUNK * NEG
NGATH = RPC // 128


def _sc_body(x_hbm, tgt_hbm, pos_hbm, neg_hbm, ent_hbm,
             pos_out, neg_out,
             tgt_v, posidx_v, negidx0, negidx1, pred_v, posrow_v,
             negrow0, negrow1, posout_v, negout0, negout1,
             isem0, isem1, rsem0, rsem1, osem0, osem1):
  nc = 2
  wid = lax.axis_index("s") * nc + lax.axis_index("c")
  base = wid * BPW
  iota = lax.iota(jnp.int32, 16)

  negidx = (negidx0, negidx1)
  negrow = (negrow0, negrow1)
  negout = (negout0, negout1)
  isem = (isem0, isem1)
  rsem = (rsem0, rsem1)
  osem = (osem0, osem1)

  def idx_slice(c):
    return neg_hbm.at[pl.ds((base + c * CHUNK) * NEG, RPC)]

  def out_slice(c):
    return neg_out.at[pl.ds((base + c * CHUNK) * NEG, RPC)]

  pltpu.sync_copy(tgt_hbm.at[pl.ds(base, BPW)], tgt_v)
  pltpu.sync_copy(pos_hbm.at[pl.ds(base, BPW)], posidx_v)
  pltpu.sync_copy(x_hbm.at[tgt_v], pred_v)
  pltpu.sync_copy(ent_hbm.at[posidx_v], posrow_v)

  def pos_group(g, carry):
    rowsg = g * 16 + iota

    def d0_body(d0, acc):
      cols = jnp.bitwise_and(iota + d0, DIM - 1)
      ev = plsc.load_gather(posrow_v, [rowsg, cols])
      pv = plsc.load_gather(pred_v, [rowsg, cols])
      return acc + jnp.abs(ev - pv)

    acc = lax.fori_loop(0, DIM, d0_body, jnp.zeros((16,), jnp.float32),
                        unroll=8)
    posout_v[pl.ds(g * 16, 16)] = GAMMA - acc
    return carry

  lax.fori_loop(0, BPW // 16, pos_group, 0)
  pltpu.sync_copy(posout_v, pos_out.at[pl.ds(base, BPW)])

  def start_rows(buf):
    for j in range(NGATH):
      pltpu.async_copy(
          ent_hbm.at[negidx[buf].at[pl.ds(j * 128, 128)]],
          negrow[buf].at[pl.ds(j * 128, 128), :],
          rsem[buf])

  def wait_rows(buf):
    for j in range(NGATH):
      pltpu.make_async_copy(
          ent_hbm.at[negidx[buf].at[pl.ds(j * 128, 128)]],
          negrow[buf].at[pl.ds(j * 128, 128), :],
          rsem[buf]).wait()

  pltpu.sync_copy(idx_slice(0), negidx[0])
  start_rows(0)
  pltpu.async_copy(idx_slice(1), negidx[1], isem[1])

  def compute_chunk(c, buf):
    for bb in range(CHUNK):
      prowv = jnp.full((16,), c * CHUNK + bb, jnp.int32)
      base_rows = bb * NEG + iota

      def d0_body(d0, accs, prowv=prowv, base_rows=base_rows, buf=buf):
        cols = jnp.bitwise_and(iota + d0, DIM - 1)
        prot = plsc.load_gather(pred_v, [prowv, cols])
        new = []
        for g in range(16):
          ev = plsc.load_gather(negrow[buf], [base_rows + g * 16, cols])
          new.append(accs[g] + jnp.abs(ev - prot))
        return tuple(new)

      accs = lax.fori_loop(0, DIM, d0_body,
                           (jnp.zeros((16,), jnp.float32),) * 16,
                           unroll=2)
      for g in range(16):
        negout[buf][pl.ds(bb * NEG + g * 16, 16)] = GAMMA - accs[g]

  def handle(c, buf):
    wait_rows(buf)

    nbuf = 1 - buf

    @pl.when(c + 1 < NCHUNKS)
    def _():
      pltpu.make_async_copy(idx_slice(c + 1), negidx[nbuf], isem[nbuf]).wait()
      start_rows(nbuf)

    @pl.when(c + 2 < NCHUNKS)
    def _():
      pltpu.async_copy(idx_slice(c + 2), negidx[buf], isem[buf])

    @pl.when(c >= 2)
    def _():
      pltpu.make_async_copy(negout[buf], out_slice(c - 2), osem[buf]).wait()

    compute_chunk(c, buf)
    pltpu.async_copy(negout[buf], out_slice(c), osem[buf])

  def pair_body(p, carry):
    handle(2 * p, 0)
    handle(2 * p + 1, 1)
    return carry

  lax.fori_loop(0, NCHUNKS // 2, pair_body, 0)

  pltpu.make_async_copy(negout[0], out_slice(NCHUNKS - 2), osem[0]).wait()
  pltpu.make_async_copy(negout[1], out_slice(NCHUNKS - 1), osem[1]).wait()


@jax.jit
def _sc_kernel(x, tgt, pos, neg_flat, ent):
  mesh = plsc.VectorSubcoreMesh(core_axis_name="c", subcore_axis_name="s")
  f = functools.partial(
      pl.kernel,
      mesh=mesh,
      compiler_params=pltpu.CompilerParams(
          needs_layout_passes=False, use_tc_tiling_on_sc=False),
      out_type=(
          jax.ShapeDtypeStruct((BATCH,), jnp.float32),
          jax.ShapeDtypeStruct((BATCH * NEG,), jnp.float32),
      ),
      scratch_types=[
          pltpu.VMEM((BPW,), jnp.int32),
          pltpu.VMEM((BPW,), jnp.int32),
          pltpu.VMEM((RPC,), jnp.int32),
          pltpu.VMEM((RPC,), jnp.int32),
          pltpu.VMEM((BPW, DIM), jnp.float32),
          pltpu.VMEM((BPW, DIM), jnp.float32),
          pltpu.VMEM((RPC, DIM), jnp.float32),
          pltpu.VMEM((RPC, DIM), jnp.float32),
          pltpu.VMEM((BPW,), jnp.float32),
          pltpu.VMEM((RPC,), jnp.float32),
          pltpu.VMEM((RPC,), jnp.float32),
          pltpu.SemaphoreType.DMA,
          pltpu.SemaphoreType.DMA,
          pltpu.SemaphoreType.DMA,
          pltpu.SemaphoreType.DMA,
          pltpu.SemaphoreType.DMA,
          pltpu.SemaphoreType.DMA,
      ],
  )(_sc_body)
  return f(x, tgt, pos, neg_flat, ent)


def kernel(x, target_node_idxes, positive_samples, negative_samples,
           ent_embedding):
  tgt = target_node_idxes.astype(jnp.int32)
  pos = positive_samples.astype(jnp.int32)
  neg_flat = negative_samples.astype(jnp.int32).reshape(-1)
  pos_l, neg_l = _sc_kernel(x, tgt, pos, neg_flat, ent_embedding)
  return pos_l[:, None], neg_l.reshape(BATCH, NEG)
